# Initial kernel scaffold; baseline (speedup 1.0000x reference)
import jax
import jax.numpy as jnp
from jax.experimental import pallas as pl


def kernel(x, edge_attr, Wh_in, We_in, Am, Bm, Cm, Dm, Em, gn, bn, ge, be, W1, b1, W2, b2, edge_index):
    raise NotImplementedError("write your pallas kernel here")


# speedup vs baseline: 2.1651x; 2.1651x over previous
import functools

import jax
import jax.numpy as jnp
from jax import lax
from jax.experimental import pallas as pl
from jax.experimental.pallas import tpu as pltpu
from jax.experimental.pallas import tpu_sc as plsc

_N = 10000
_NE = 320000
_H = 128
_L = 4
_MAX_ACTION = 1.0

_NC = 2
_NS = 16
_NW = _NC * _NS

_F32 = jnp.float32

_GC = 80
_PER_W = _NE // _NW
_G_CHUNKS = _PER_W // _GC


def _gather3_body(tab_d, tab_e, tab_b, src_hbm, dst_hbm, g_d, g_e, g_b,
                  idx_v, rows_v, sem):
    wid = lax.axis_index("s") * _NC + lax.axis_index("c")
    base0 = wid * _PER_W

    def step(i, carry):
        base = base0 + i * _GC
        pltpu.sync_copy(src_hbm.at[pl.ds(base, _GC)], idx_v)
        pltpu.async_copy(tab_d.at[idx_v], rows_v, sem).wait()
        pltpu.sync_copy(rows_v, g_d.at[pl.ds(base, _GC)])
        pltpu.async_copy(tab_b.at[idx_v], rows_v, sem).wait()
        pltpu.sync_copy(rows_v, g_b.at[pl.ds(base, _GC)])
        pltpu.sync_copy(dst_hbm.at[pl.ds(base, _GC)], idx_v)
        pltpu.async_copy(tab_e.at[idx_v], rows_v, sem).wait()
        pltpu.sync_copy(rows_v, g_e.at[pl.ds(base, _GC)])
        return carry

    lax.fori_loop(0, _G_CHUNKS, step, 0)


_gather3 = pl.kernel(
    _gather3_body,
    out_type=[jax.ShapeDtypeStruct((_NE, _H), _F32)] * 3,
    mesh=plsc.VectorSubcoreMesh(core_axis_name="c", subcore_axis_name="s"),
    scratch_types=[
        pltpu.VMEM((_GC,), jnp.int32),
        pltpu.VMEM((_GC, _H), _F32),
        pltpu.SemaphoreType.DMA,
    ],
)

_S_PER_CORE = _NE // _NC
_S_PER_TILE = _S_PER_CORE // _NS
_S_CHUNKS = _S_PER_TILE // _GC
_Z_CHUNKS = _N // _GC


def _segsum_body(vals, dst_hbm, zeros_hbm, out, idx_v, vals_v, accum_sh, sem):
    cid = lax.axis_index("c")
    sid = lax.axis_index("s")

    pltpu.sync_copy(zeros_hbm, vals_v)

    def zstep(j, carry):
        @pl.when(lax.rem(j, _NS) == sid)
        def _():
            pltpu.sync_copy(vals_v, accum_sh.at[pl.ds(j * _GC, _GC)])
        return carry

    lax.fori_loop(0, _Z_CHUNKS, zstep, 0)
    plsc.subcore_barrier()

    base0 = cid * _S_PER_CORE + sid * _S_PER_TILE

    def step(i, carry):
        base = base0 + i * _GC
        pltpu.sync_copy(dst_hbm.at[pl.ds(base, _GC)], idx_v)
        pltpu.sync_copy(vals.at[pl.ds(base, _GC)], vals_v)
        pltpu.sync_copy(vals_v, accum_sh.at[idx_v], add=True)
        return carry

    lax.fori_loop(0, _S_CHUNKS, step, 0)
    plsc.subcore_barrier()

    def wstep(j, carry):
        @pl.when(lax.rem(j, _NS) == sid)
        def _():
            pltpu.sync_copy(accum_sh.at[pl.ds(j * _GC, _GC)], vals_v)
            pltpu.sync_copy(vals_v, out.at[cid, pl.ds(j * _GC, _GC)])
        return carry

    lax.fori_loop(0, _Z_CHUNKS, wstep, 0)


_segsum = pl.kernel(
    _segsum_body,
    out_type=jax.ShapeDtypeStruct((_NC, _N, _H), _F32),
    mesh=plsc.VectorSubcoreMesh(core_axis_name="c", subcore_axis_name="s"),
    scratch_types=[
        pltpu.VMEM((_GC,), jnp.int32),
        pltpu.VMEM((_GC, _H), _F32),
        pltpu.VMEM_SHARED((_N, _H), _F32),
        pltpu.SemaphoreType.DMA,
    ],
)

_BE = 4000
_EGRID = _NE // _BE


def _in_proj_body(x_ref, wh_ref, ea_ref, we_ref, h_ref, e_ref):
    i = pl.program_id(0)

    @pl.when(i == 0)
    def _():
        h_ref[...] = jnp.dot(x_ref[...], wh_ref[...],
                             preferred_element_type=_F32)

    e_ref[...] = jnp.dot(ea_ref[...], we_ref[...],
                         preferred_element_type=_F32)


def _in_proj(x, wh, ea, we):
    return pl.pallas_call(
        _in_proj_body,
        grid=(_EGRID,),
        in_specs=[
            pl.BlockSpec((_N, _H), lambda i: (0, 0)),
            pl.BlockSpec((_H, _H), lambda i: (0, 0)),
            pl.BlockSpec((_BE, 16), lambda i: (i, 0)),
            pl.BlockSpec((16, _H), lambda i: (0, 0)),
        ],
        out_specs=[
            pl.BlockSpec((_N, _H), lambda i: (0, 0)),
            pl.BlockSpec((_BE, _H), lambda i: (i, 0)),
        ],
        out_shape=[
            jax.ShapeDtypeStruct((_N, _H), _F32),
            jax.ShapeDtypeStruct((_NE, _H), _F32),
        ],
    )(x, wh, ea, we)


def _node_mm_body(h_ref, wa, wb, wd, we, oa, ob, od, oe):
    h = h_ref[...]
    oa[...] = jnp.dot(h, wa[...], preferred_element_type=_F32)
    ob[...] = jnp.dot(h, wb[...], preferred_element_type=_F32)
    od[...] = jnp.dot(h, wd[...], preferred_element_type=_F32)
    oe[...] = jnp.dot(h, we[...], preferred_element_type=_F32)


def _node_mm(h, wa, wb, wd, we):
    shp = jax.ShapeDtypeStruct((_N, _H), _F32)
    return pl.pallas_call(_node_mm_body, out_shape=[shp] * 4)(h, wa, wb, wd, we)


def _edge_main_body(e_ref, gd_ref, ge_ref, gb_ref, c_ref,
                    ehat_ref, sig_ref, p_ref, stats_ref):
    i = pl.program_id(0)
    ec = jnp.dot(e_ref[...], c_ref[...], preferred_element_type=_F32)
    ehat = ec + gd_ref[...] + ge_ref[...]
    sig = 1.0 / (1.0 + jnp.exp(-ehat))
    ehat_ref[...] = ehat
    sig_ref[...] = sig
    p_ref[...] = sig * gb_ref[...]
    s1 = jnp.sum(ehat, axis=0, keepdims=True)
    s2 = jnp.sum(ehat * ehat, axis=0, keepdims=True)
    st = jnp.concatenate([s1, s2], axis=0)

    @pl.when(i == 0)
    def _():
        stats_ref[...] = st

    @pl.when(i > 0)
    def _():
        stats_ref[...] = stats_ref[...] + st


def _edge_main(e, gd, ge, gb, cm):
    eb = pl.BlockSpec((_BE, _H), lambda i: (i, 0))
    return pl.pallas_call(
        _edge_main_body,
        grid=(_EGRID,),
        in_specs=[eb, eb, eb, eb, pl.BlockSpec((_H, _H), lambda i: (0, 0))],
        out_specs=[eb, eb, eb, pl.BlockSpec((2, _H), lambda i: (0, 0))],
        out_shape=[
            jax.ShapeDtypeStruct((_NE, _H), _F32),
            jax.ShapeDtypeStruct((_NE, _H), _F32),
            jax.ShapeDtypeStruct((_NE, _H), _F32),
            jax.ShapeDtypeStruct((2, _H), _F32),
        ],
    )(e, gd, ge, gb, cm)


def _node_update_body(h_ref, ah_ref, nump_ref, denp_ref, g_ref, b_ref, o_ref):
    num = nump_ref[0] + nump_ref[1]
    den = denp_ref[0] + denp_ref[1] + 1e-6
    h_hat = ah_ref[...] + num / den
    mu = jnp.mean(h_hat, axis=0, keepdims=True)
    var = jnp.mean(h_hat * h_hat, axis=0, keepdims=True) - mu * mu
    hn = g_ref[...] * (h_hat - mu) / jnp.sqrt(var + 1e-5) + b_ref[...]
    o_ref[...] = h_ref[...] + jnp.maximum(hn, 0.0)


def _node_update(h, ah, nump, denp, g, b):
    return pl.pallas_call(
        _node_update_body,
        out_shape=jax.ShapeDtypeStruct((_N, _H), _F32),
    )(h, ah, nump, denp, g, b)


def _edge_update_body(e_ref, ehat_ref, stats_ref, g_ref, b_ref, o_ref):
    st = stats_ref[...]
    mu = st[0:1] * (1.0 / _NE)
    var = st[1:2] * (1.0 / _NE) - mu * mu
    en = g_ref[...] * (ehat_ref[...] - mu) / jnp.sqrt(var + 1e-5) + b_ref[...]
    o_ref[...] = e_ref[...] + jnp.maximum(en, 0.0)


def _edge_update(e, ehat, stats, g, b):
    eb = pl.BlockSpec((_BE, _H), lambda i: (i, 0))
    cb = lambda shp: pl.BlockSpec(shp, lambda i: (0, 0))
    return pl.pallas_call(
        _edge_update_body,
        grid=(_EGRID,),
        in_specs=[eb, eb, cb((2, _H)), cb((1, _H)), cb((1, _H))],
        out_specs=eb,
        out_shape=jax.ShapeDtypeStruct((_NE, _H), _F32),
    )(e, ehat, stats, g, b)


def _readout_body(h_ref, w1_ref, b1_ref, w2_ref, b2_ref, o_ref):
    hg = jnp.mean(h_ref[...], axis=0, keepdims=True)
    z = jnp.dot(hg, w1_ref[...], preferred_element_type=_F32) + b1_ref[...]
    z = jnp.maximum(z, 0.0)
    o = jnp.dot(z, w2_ref[...], preferred_element_type=_F32) + b2_ref[...]
    o_ref[...] = _MAX_ACTION * jnp.tanh(o)


def _readout(h, w1, b1, w2, b2):
    return pl.pallas_call(
        _readout_body,
        out_shape=jax.ShapeDtypeStruct((1, 8), _F32),
    )(h, w1, b1, w2, b2)


def kernel(x, edge_attr, Wh_in, We_in, Am, Bm, Cm, Dm, Em, gn, bn, ge, be,
           W1, b1, W2, b2, edge_index):
    src = edge_index[0]
    dst = edge_index[1]
    zeros_chunk = jnp.zeros((_GC, _H), _F32)

    h, e = _in_proj(x, Wh_in, edge_attr, We_in)

    for l in range(_L):
        ah, bh, dh, eh = _node_mm(h, Am[l], Bm[l], Dm[l], Em[l])
        gd, ge_g, gb = _gather3(dh, eh, bh, src, dst)
        ehat, sig, p, estats = _edge_main(e, gd, ge_g, gb, Cm[l])
        nump = _segsum(p, dst, zeros_chunk)
        denp = _segsum(sig, dst, zeros_chunk)
        h = _node_update(h, ah, nump, denp,
                         gn[l].reshape(1, _H), bn[l].reshape(1, _H))
        if l < _L - 1:
            e = _edge_update(e, ehat, estats,
                             ge[l].reshape(1, _H), be[l].reshape(1, _H))

    return _readout(h, W1, b1.reshape(1, -1), W2, b2.reshape(1, -1))


# speedup vs baseline: 2.4988x; 1.1541x over previous
import functools

import jax
import jax.numpy as jnp
from jax import lax
from jax.experimental import pallas as pl
from jax.experimental.pallas import tpu as pltpu
from jax.experimental.pallas import tpu_sc as plsc

_N = 10000
_NE = 320000
_H = 128
_HH = 64
_L = 4
_MAX_ACTION = 1.0

_NC = 2
_NS = 16
_F32 = jnp.float32

_SUB = 128
_IDR = 4
_BC = _SUB * _IDR
_NG = _NE // _BC
_ZCH = 80
_NZCH = _N // _ZCH


def _sc_half(e_off, src_rs, dst_rs, ec_c, db_c, ehf, ehat_c, nd_c,
             st_c, idxs_v, idxd_v, a_v, db_v, eg_v, st_v, sem,
             nd_sh, sid, write_ehat):
    def chunk(t, stat):
        g = sid + _NS * t
        base = g * _BC
        pltpu.sync_copy(src_rs.at[g], idxs_v)
        pltpu.sync_copy(dst_rs.at[g], idxd_v)
        for j in range(_IDR):
            hbase = base + j * _SUB
            pltpu.sync_copy(ec_c.at[pl.ds(hbase, _SUB)], a_v)
            cp1 = pltpu.async_copy(db_c.at[idxs_v.at[j]], db_v, sem)
            cp2 = pltpu.async_copy(ehf.at[idxd_v.at[j]], eg_v, sem)
            cp1.wait()
            cp2.wait()

            def rowfn(r, stat_r):
                s1s, s2s = stat_r
                n1, n2 = [], []
                for k in range(_HH // 16):
                    sl = pl.ds(k * 16, 16)
                    sld = pl.ds(e_off + k * 16, 16)
                    slb = pl.ds(_HH + k * 16, 16)
                    eh = a_v[r, sl] + db_v[r, sl] + eg_v[r, sld]
                    a_v[r, sl] = eh
                    sig = 1.0 / (1.0 + jnp.exp(-eh))
                    db_v[r, slb] = db_v[r, slb] * sig
                    db_v[r, sl] = sig
                    n1.append(s1s[k] + eh)
                    n2.append(s2s[k] + eh * eh)
                return (tuple(n1), tuple(n2))

            stat = lax.fori_loop(0, _SUB, rowfn, stat)
            if write_ehat:
                pltpu.sync_copy(a_v, ehat_c.at[pl.ds(hbase, _SUB)])
            pltpu.sync_copy(db_v, nd_sh.at[idxd_v.at[j]], add=True)
        return stat

    zero = tuple(jnp.zeros((16,), _F32) for _ in range(_HH // 16))
    ngroups = (_NG // _NS) + jnp.where(sid < (_NG % _NS), 1, 0)
    s1s, s2s = lax.fori_loop(0, ngroups, chunk, (zero, zero))
    for k in range(_HH // 16):
        sl = pl.ds(k * 16, 16)
        st_v[0, sl] = s1s[k]
        st_v[1, sl] = s2s[k]
    pltpu.sync_copy(st_v, st_c.at[sid])
    plsc.subcore_barrier()

    def drain(j, c):
        @pl.when(lax.rem(j, _NS) == sid)
        def _():
            sl = pl.ds(j * _ZCH, _ZCH)
            bsl = pl.ds(0, _ZCH)
            pltpu.sync_copy(nd_sh.at[sl], db_v.at[bsl])
            pltpu.sync_copy(db_v.at[bsl], nd_c.at[sl])
        return c

    lax.fori_loop(0, _NZCH, drain, 0)


def _make_edge_sc(write_ehat):
    def body(ec0, ec1, db0, db1, ehf, src_rs, dst_rs, zrows,
             ehat0, ehat1, nd0, nd1, st0, st1,
             idxs_v, idxd_v, a_v, db_v, eg_v, st_v, sem, nd_sh):
        cid = lax.axis_index("c")
        sid = lax.axis_index("s")
        pltpu.sync_copy(zrows, db_v.at[pl.ds(0, _ZCH)])

        def zstep(j, c):
            @pl.when(lax.rem(j, _NS) == sid)
            def _():
                sl = pl.ds(j * _ZCH, _ZCH)
                pltpu.sync_copy(db_v.at[pl.ds(0, _ZCH)], nd_sh.at[sl])
            return c

        lax.fori_loop(0, _NZCH, zstep, 0)
        plsc.subcore_barrier()

        @pl.when(cid == 0)
        def _():
            _sc_half(0, src_rs, dst_rs, ec0, db0, ehf, ehat0, nd0,
                     st0, idxs_v, idxd_v, a_v, db_v, eg_v, st_v, sem,
                     nd_sh, sid, write_ehat)

        @pl.when(cid == 1)
        def _():
            _sc_half(_HH, src_rs, dst_rs, ec1, db1, ehf, ehat1, nd1,
                     st1, idxs_v, idxd_v, a_v, db_v, eg_v, st_v, sem,
                     nd_sh, sid, write_ehat)

    return pl.kernel(
        body,
        out_type=[
            jax.ShapeDtypeStruct((_NE, _HH), _F32),
            jax.ShapeDtypeStruct((_NE, _HH), _F32),
            jax.ShapeDtypeStruct((_N, _H), _F32),
            jax.ShapeDtypeStruct((_N, _H), _F32),
            jax.ShapeDtypeStruct((_NS, 2, _HH), _F32),
            jax.ShapeDtypeStruct((_NS, 2, _HH), _F32),
        ],
        mesh=plsc.VectorSubcoreMesh(core_axis_name="c", subcore_axis_name="s"),
        scratch_types=[
            pltpu.VMEM((_IDR, _SUB), jnp.int32),
            pltpu.VMEM((_IDR, _SUB), jnp.int32),
            pltpu.VMEM((_SUB, _HH), _F32),
            pltpu.VMEM((_SUB, _H), _F32),
            pltpu.VMEM((_SUB, _H), _F32),
            pltpu.VMEM((2, _HH), _F32),
            pltpu.SemaphoreType.DMA,
            pltpu.VMEM_SHARED((_N, _H), _F32),
        ],
    )


_edge_sc = _make_edge_sc(True)
_edge_sc_last = _make_edge_sc(False)

_BE = 4000
_EGRID = _NE // _BE


def _in_proj_body(x_ref, wh_ref, ea_ref, we_ref, cm_ref,
                  h_ref, e_ref, ec0_ref, ec1_ref):
    i = pl.program_id(0)

    @pl.when(i == 0)
    def _():
        h_ref[...] = jnp.dot(x_ref[...], wh_ref[...],
                             preferred_element_type=_F32)

    e0 = jnp.dot(ea_ref[...], we_ref[...], preferred_element_type=_F32)
    e_ref[...] = e0
    ec = jnp.dot(e0, cm_ref[...], preferred_element_type=_F32)
    ec0_ref[...] = ec[:, :_HH]
    ec1_ref[...] = ec[:, _HH:]


def _in_proj(x, wh, ea, we, cm0):
    ebf = pl.BlockSpec((_BE, _H), lambda i: (i, 0))
    ebh = pl.BlockSpec((_BE, _HH), lambda i: (i, 0))
    return pl.pallas_call(
        _in_proj_body,
        grid=(_EGRID,),
        in_specs=[
            pl.BlockSpec((_N, _H), lambda i: (0, 0)),
            pl.BlockSpec((_H, _H), lambda i: (0, 0)),
            pl.BlockSpec((_BE, 16), lambda i: (i, 0)),
            pl.BlockSpec((16, _H), lambda i: (0, 0)),
            pl.BlockSpec((_H, _H), lambda i: (0, 0)),
        ],
        out_specs=[pl.BlockSpec((_N, _H), lambda i: (0, 0)), ebf, ebh, ebh],
        out_shape=[
            jax.ShapeDtypeStruct((_N, _H), _F32),
            jax.ShapeDtypeStruct((_NE, _H), _F32),
            jax.ShapeDtypeStruct((_NE, _HH), _F32),
            jax.ShapeDtypeStruct((_NE, _HH), _F32),
        ],
    )(x, wh, ea, we, cm0)


def _node_mm_body(h_ref, wa, wdb0, wdb1, wem, oa, odb0, odb1, oe):
    h = h_ref[...]
    oa[...] = jnp.dot(h, wa[...], preferred_element_type=_F32)
    odb0[...] = jnp.dot(h, wdb0[...], preferred_element_type=_F32)
    odb1[...] = jnp.dot(h, wdb1[...], preferred_element_type=_F32)
    oe[...] = jnp.dot(h, wem[...], preferred_element_type=_F32)


def _node_mm(h, wa, wdb0, wdb1, wem):
    full = jax.ShapeDtypeStruct((_N, _H), _F32)
    return pl.pallas_call(
        _node_mm_body,
        out_shape=[full, full, full, full],
    )(h, wa, wdb0, wdb1, wem)


def _edge_fused_body(e_ref, eh0_ref, eh1_ref, st0_ref, st1_ref,
                     g_ref, b_ref, cm_ref, enew_ref, ec0_ref, ec1_ref):
    ss0 = jnp.sum(st0_ref[...], axis=0)
    ss1 = jnp.sum(st1_ref[...], axis=0)
    mu = jnp.concatenate([ss0[0:1], ss1[0:1]], axis=1) * (1.0 / _NE)
    ms = jnp.concatenate([ss0[1:2], ss1[1:2]], axis=1) * (1.0 / _NE)
    var = ms - mu * mu
    inv = g_ref[...] / jnp.sqrt(var + 1e-5)
    ehat = jnp.concatenate([eh0_ref[...], eh1_ref[...]], axis=1)
    en = (ehat - mu) * inv + b_ref[...]
    enew = e_ref[...] + jnp.maximum(en, 0.0)
    enew_ref[...] = enew
    ec = jnp.dot(enew, cm_ref[...], preferred_element_type=_F32)
    ec0_ref[...] = ec[:, :_HH]
    ec1_ref[...] = ec[:, _HH:]


def _edge_fused(e, eh0, eh1, st0, st1, g, b, cm):
    ebf = pl.BlockSpec((_BE, _H), lambda i: (i, 0))
    ebh = pl.BlockSpec((_BE, _HH), lambda i: (i, 0))
    cb = lambda shp: pl.BlockSpec(shp, lambda i: (0,) * len(shp))
    return pl.pallas_call(
        _edge_fused_body,
        grid=(_EGRID,),
        in_specs=[ebf, ebh, ebh, cb((_NS, 2, _HH)), cb((_NS, 2, _HH)),
                  cb((1, _H)), cb((1, _H)), cb((_H, _H))],
        out_specs=[ebf, ebh, ebh],
        out_shape=[
            jax.ShapeDtypeStruct((_NE, _H), _F32),
            jax.ShapeDtypeStruct((_NE, _HH), _F32),
            jax.ShapeDtypeStruct((_NE, _HH), _F32),
        ],
    )(e, eh0, eh1, st0, st1, g, b, cm)


def _node_update_body(h_ref, ah_ref, nd0_ref, nd1_ref, g_ref, b_ref, o_ref):
    nd0 = nd0_ref[...]
    nd1 = nd1_ref[...]
    num = jnp.concatenate([nd0[:, _HH:], nd1[:, _HH:]], axis=1)
    den = jnp.concatenate([nd0[:, :_HH], nd1[:, :_HH]], axis=1) + 1e-6
    h_hat = ah_ref[...] + num / den
    mu = jnp.mean(h_hat, axis=0, keepdims=True)
    var = jnp.mean(h_hat * h_hat, axis=0, keepdims=True) - mu * mu
    hn = g_ref[...] * (h_hat - mu) / jnp.sqrt(var + 1e-5) + b_ref[...]
    o_ref[...] = h_ref[...] + jnp.maximum(hn, 0.0)


def _node_update(h, ah, nd0, nd1, g, b):
    return pl.pallas_call(
        _node_update_body,
        out_shape=jax.ShapeDtypeStruct((_N, _H), _F32),
    )(h, ah, nd0, nd1, g, b)


def _readout_body(h_ref, w1_ref, b1_ref, w2_ref, b2_ref, o_ref):
    hg = jnp.mean(h_ref[...], axis=0, keepdims=True)
    z = jnp.dot(hg, w1_ref[...], preferred_element_type=_F32) + b1_ref[...]
    z = jnp.maximum(z, 0.0)
    o = jnp.dot(z, w2_ref[...], preferred_element_type=_F32) + b2_ref[...]
    o_ref[...] = _MAX_ACTION * jnp.tanh(o)


def _readout(h, w1, b1, w2, b2):
    return pl.pallas_call(
        _readout_body,
        out_shape=jax.ShapeDtypeStruct((1, 8), _F32),
    )(h, w1, b1, w2, b2)


def kernel(x, edge_attr, Wh_in, We_in, Am, Bm, Cm, Dm, Em, gn, bn, ge, be,
           W1, b1, W2, b2, edge_index):
    src_rs = edge_index[0].reshape(_NG, _IDR, _SUB)
    dst_rs = edge_index[1].reshape(_NG, _IDR, _SUB)
    zrows = jnp.zeros((_ZCH, _H), _F32)

    h, e, ec0, ec1 = _in_proj(x, Wh_in, edge_attr, We_in, Cm[0])
    eh0 = eh1 = st0 = st1 = None

    for l in range(_L):
        wdb0 = jnp.concatenate([Dm[l][:, :_HH], Bm[l][:, :_HH]], axis=1)
        wdb1 = jnp.concatenate([Dm[l][:, _HH:], Bm[l][:, _HH:]], axis=1)
        ah, db0, db1, ehf = _node_mm(h, Am[l], wdb0, wdb1, Em[l])
        if l > 0:
            e, ec0, ec1 = _edge_fused(e, eh0, eh1, st0, st1,
                                      ge[l - 1].reshape(1, _H),
                                      be[l - 1].reshape(1, _H), Cm[l])
        sc = _edge_sc if l < _L - 1 else _edge_sc_last
        eh0, eh1, nd0, nd1, st0, st1 = sc(
            ec0, ec1, db0, db1, ehf, src_rs, dst_rs, zrows)
        h = _node_update(h, ah, nd0, nd1,
                         gn[l].reshape(1, _H), bn[l].reshape(1, _H))

    return _readout(h, W1, b1.reshape(1, -1), W2, b2.reshape(1, -1))
